# baseline (speedup 1.0000x reference)
import jax
import jax.numpy as jnp
from jax import lax
from jax.experimental import pallas as pl
from jax.experimental.pallas import tpu as pltpu
from jax.experimental.pallas import tpu_sc as plsc

N = 10000
D = 128
G = 64
E = 320000

NP = 10112
TILES = 16
K = 88
T = 228
EP = TILES * T * K
NROW = 2 * EP // K
TROW = T
NPT = NP // TILES
BLK = 632
NBLK = (2 * NP) // BLK

ROWB = K * D * 4
IDXB = K * 4


def _memset_rows(ref, rows, cols, val):
    v = jnp.full((16,), val, jnp.float32)

    def body(i, _):
        r = i // (cols // 16)
        c = i % (cols // 16)
        ref[r, pl.ds(c * 16, 16)] = v
        return 0

    lax.fori_loop(0, rows * (cols // 16), body, 0)


def _sc_mesh():
    return plsc.VectorSubcoreMesh(core_axis_name="c", subcore_axis_name="s")


def _deg_body(dst_hbm, deg_hbm, shared_deg, didx, ones_v, zero_v,
              is0, is1, is2, is3, is4, is5):
    c = lax.axis_index("c")
    s = lax.axis_index("s")
    isem = [is0, is1, is2, is3, is4, is5]

    _memset_rows(ones_v, K, 16, 1.0)
    _memset_rows(zero_v, NPT, 16, 0.0)
    pltpu.sync_copy(zero_v, shared_deg.at[pl.ds(s * NPT, NPT)])
    plsc.subcore_barrier()

    base = c * EP + s * TROW * K

    def load_idx(j, u):
        pltpu.async_copy(dst_hbm.at[pl.ds(base + j * K, K)], didx.at[u],
                         isem[u])

    def wait_idx(u):
        pltpu.make_async_copy(dst_hbm.at[pl.ds(0, K)], didx.at[u],
                              isem[u]).wait()

    for j in range(4):
        load_idx(j, j % 6)

    def body(it, _):
        j0 = it * 6
        for u in range(6):
            j = j0 + u
            wait_idx(u)
            pltpu.sync_copy(ones_v, shared_deg.at[didx.at[u]], add=True)

            @pl.when(j + 4 < T)
            def _():
                load_idx(j + 4, (u + 4) % 6)
        return 0

    lax.fori_loop(0, T // 6, body, 0)
    plsc.subcore_barrier()

    pltpu.sync_copy(shared_deg.at[pl.ds(s * NPT, NPT)],
                    deg_hbm.at[pl.ds(c * NP + s * NPT, NPT)])


_deg_call = pl.kernel(
    _deg_body,
    out_type=jax.ShapeDtypeStruct((2 * NP, 16), jnp.float32),
    mesh=_sc_mesh(),
    scratch_types=[
        pltpu.VMEM_SHARED((NP, 16), jnp.float32),
        pltpu.VMEM((6, K), jnp.int32),
        pltpu.VMEM((K, 16), jnp.float32),
        pltpu.VMEM((NPT, 16), jnp.float32),
    ] + [pltpu.SemaphoreType.DMA] * 6,
)


def _agg_body(g_hbm, src_hbm, dst_hbm, agg_hbm,
              shared_agg, rows0, rows1, rows2, rows3, sidx, didx,
              gs0, gs1, gs2, gs3, ss0, ss1, ss2, ss3,
              is0, is1, is2, is3, is4, is5,
              js0, js1, js2, js3, js4, js5):
    c = lax.axis_index("c")
    s = lax.axis_index("s")
    rows = [rows0, rows1, rows2, rows3]
    gsem = [gs0, gs1, gs2, gs3]
    ssem = [ss0, ss1, ss2, ss3]
    isem = [is0, is1, is2, is3, is4, is5]
    jsem = [js0, js1, js2, js3, js4, js5]

    _memset_rows(rows0, K, D, 0.0)
    for t in range(7):
        pltpu.sync_copy(rows0, shared_agg.at[pl.ds(s * NPT + t * K, K)])
    pltpu.sync_copy(rows0, shared_agg.at[pl.ds(s * NPT + NPT - K, K)])
    plsc.subcore_barrier()

    base = c * EP + s * TROW * K

    def load_idx(j, u):
        pltpu.async_copy(src_hbm.at[pl.ds(base + j * K, K)], sidx.at[u],
                         isem[u])
        pltpu.async_copy(dst_hbm.at[pl.ds(base + j * K, K)], didx.at[u],
                         jsem[u])

    def wait_idx(u):
        pltpu.make_async_copy(src_hbm.at[pl.ds(0, K)], sidx.at[u],
                              isem[u]).wait()
        pltpu.make_async_copy(src_hbm.at[pl.ds(0, K)], didx.at[u],
                              jsem[u]).wait()

    def start_gather(u, u3):
        pltpu.async_copy(g_hbm.at[sidx.at[u]], rows[u3], gsem[u3])

    def wait_gather(u3):
        pltpu.make_async_copy(g_hbm.at[pl.ds(0, K)], rows[u3],
                              gsem[u3]).wait()

    def start_scatter(u, u3):
        pltpu.async_copy(rows[u3], shared_agg.at[didx.at[u]],
                         ssem[u3], add=True)

    def wait_scatter(u3):
        pltpu.make_async_copy(g_hbm.at[pl.ds(0, K)], rows[u3],
                              ssem[u3]).wait()

    for j in range(5):
        load_idx(j, j)
    for j in range(3):
        wait_idx(j)
        start_gather(j, j)

    def body(it, _):
        j0 = it * 12
        for u in range(12):
            j = j0 + u

            @pl.when(j >= 1)
            def _():
                wait_scatter((u + 3) % 4)

            @pl.when(j + 3 < T)
            def _():
                wait_idx((u + 3) % 6)
                start_gather((u + 3) % 6, (u + 3) % 4)

            @pl.when(j + 5 < T)
            def _():
                load_idx(j + 5, (u + 5) % 6)

            wait_gather(u % 4)
            start_scatter(u % 6, u % 4)
        return 0

    lax.fori_loop(0, T // 12, body, 0)
    wait_scatter((T - 1) % 4)
    plsc.subcore_barrier()

    pltpu.sync_copy(shared_agg.at[pl.ds(s * NPT, NPT)],
                    agg_hbm.at[pl.ds(c * NP + s * NPT, NPT)])


_agg_call = pl.kernel(
    _agg_body,
    out_type=jax.ShapeDtypeStruct((2 * NP, D), jnp.float32),
    mesh=_sc_mesh(),
    scratch_types=[
        pltpu.VMEM_SHARED((NP, D), jnp.float32),
        pltpu.VMEM((K, D), jnp.float32),
        pltpu.VMEM((K, D), jnp.float32),
        pltpu.VMEM((K, D), jnp.float32),
        pltpu.VMEM((K, D), jnp.float32),
        pltpu.VMEM((6, K), jnp.int32),
        pltpu.VMEM((6, K), jnp.int32),
    ] + [pltpu.SemaphoreType.DMA] * 20,
)


def _dis_block(deg_ref):
    d0 = deg_ref[...][:, 0:1] + 1.0
    return lax.rsqrt(d0)


def _mm_body(x_ref, w_ref, m_ref):
    m_ref[...] = jnp.dot(x_ref[...], w_ref[...],
                         preferred_element_type=jnp.float32)


_mm_call = pl.pallas_call(
    _mm_body,
    grid=(NBLK,),
    in_specs=[
        pl.BlockSpec((BLK, D), lambda i: (i, 0)),
        pl.BlockSpec((D, D), lambda i: (0, 0)),
    ],
    out_specs=pl.BlockSpec((BLK, D), lambda i: (i, 0)),
    out_shape=jax.ShapeDtypeStruct((2 * NP, D), jnp.float32),
)


def _scale_body(m_ref, deg_ref, g_ref):
    g_ref[...] = m_ref[...] * _dis_block(deg_ref)


_scale_call = pl.pallas_call(
    _scale_body,
    grid=(NBLK,),
    in_specs=[
        pl.BlockSpec((BLK, D), lambda i: (i, 0)),
        pl.BlockSpec((BLK, 16), lambda i: (i, 0)),
    ],
    out_specs=pl.BlockSpec((BLK, D), lambda i: (i, 0)),
    out_shape=jax.ShapeDtypeStruct((2 * NP, D), jnp.float32),
)


def _layer_body(agg_ref, g_ref, deg_ref, w_ref, b_ref, out_ref):
    dis = _dis_block(deg_ref)
    h = (agg_ref[...] + g_ref[...]) * dis + b_ref[...]
    h = jnp.maximum(h, 0.0)
    out_ref[...] = jnp.dot(h, w_ref[...],
                           preferred_element_type=jnp.float32) * dis


_layer_call = pl.pallas_call(
    _layer_body,
    grid=(NBLK,),
    in_specs=[
        pl.BlockSpec((BLK, D), lambda i: (i, 0)),
        pl.BlockSpec((BLK, D), lambda i: (i, 0)),
        pl.BlockSpec((BLK, 16), lambda i: (i, 0)),
        pl.BlockSpec((D, D), lambda i: (0, 0)),
        pl.BlockSpec((1, D), lambda i: (0, 0)),
    ],
    out_specs=pl.BlockSpec((BLK, D), lambda i: (i, 0)),
    out_shape=jax.ShapeDtypeStruct((2 * NP, D), jnp.float32),
)


def _final_body(agg_ref, g_ref, deg_ref, b_ref, batch_ref, wl_ref, bl_ref,
                out_ref, acc, cnt):
    i = pl.program_id(0)

    @pl.when(i == 0)
    def _():
        acc[...] = jnp.zeros_like(acc)
        cnt[...] = jnp.zeros_like(cnt)

    dis = _dis_block(deg_ref)
    h = (agg_ref[...] + g_ref[...]) * dis + b_ref[...]

    b2 = batch_ref[...].reshape(1, BLK)
    onehot = (lax.broadcasted_iota(jnp.int32, (G, BLK), 0) == b2)
    onehot = onehot.astype(jnp.float32)

    part = jnp.dot(onehot, h, preferred_element_type=jnp.float32)
    c = i // (NBLK // 2)
    acc[:, pl.ds(c * D, D)] += part
    cnt[:, pl.ds(c * D, D)] += jnp.sum(onehot, axis=1, keepdims=True)

    @pl.when(i == NBLK - 1)
    def _():
        pooled = acc[...] / jnp.maximum(cnt[...], 1.0)
        out_ref[...] = jnp.dot(pooled, wl_ref[...],
                               preferred_element_type=jnp.float32) + bl_ref[...]


_final_call = pl.pallas_call(
    _final_body,
    grid=(NBLK,),
    in_specs=[
        pl.BlockSpec((BLK, D), lambda i: (i, 0)),
        pl.BlockSpec((BLK, D), lambda i: (i, 0)),
        pl.BlockSpec((BLK, 16), lambda i: (i, 0)),
        pl.BlockSpec((1, D), lambda i: (0, 0)),
        pl.BlockSpec((1, 1, BLK), lambda i: (i, 0, 0)),
        pl.BlockSpec((2 * D, 2), lambda i: (0, 0)),
        pl.BlockSpec((1, 2), lambda i: (0, 0)),
    ],
    out_specs=pl.BlockSpec((G, 2), lambda i: (0, 0)),
    out_shape=jax.ShapeDtypeStruct((G, 2), jnp.float32),
    scratch_shapes=[
        pltpu.VMEM((G, 2 * D), jnp.float32),
        pltpu.VMEM((G, 2 * D), jnp.float32),
    ],
)


def kernel(x0, edge_attr0, edge_index0, x1, edge_attr1, edge_index1,
           batch0, batch1, W1, b1, W2, b2, W3, b3, WL, bL):
    npad = EP - E
    pad_rows = N + (jnp.arange(npad, dtype=jnp.int32) % (NP - N))

    src0 = jnp.concatenate([edge_index0[0], pad_rows])
    dst0 = jnp.concatenate([edge_index0[1], pad_rows])
    src1 = jnp.concatenate([edge_index1[0], pad_rows]) + NP
    dst1 = jnp.concatenate([edge_index1[1], pad_rows])
    src_all = jnp.concatenate([src0, src1])
    dst_all = jnp.concatenate([dst0, dst1])

    zrows = jnp.zeros((NP - N, D), jnp.float32)
    x_pad = jnp.concatenate([x0, zrows, x1, zrows])

    bpad = jnp.full((NP - N,), G, jnp.int32)
    batch_all = jnp.concatenate([batch0, bpad, batch1, bpad])
    batch3d = batch_all.reshape(NBLK, 1, BLK)

    b1r = b1.reshape(1, D)
    b2r = b2.reshape(1, D)
    b3r = b3.reshape(1, D)
    bLr = bL.reshape(1, 2)

    deg16 = _deg_call(dst_all)
    m1 = _mm_call(x_pad, W1)

    g1 = _scale_call(m1, deg16)
    agg1 = _agg_call(g1, src_all, dst_all)
    g2 = _layer_call(agg1, g1, deg16, W2, b1r)
    agg2 = _agg_call(g2, src_all, dst_all)
    g3 = _layer_call(agg2, g2, deg16, W3, b2r)
    agg3 = _agg_call(g3, src_all, dst_all)
    return _final_call(agg3, g3, deg16, b3r, batch3d, WL, bLr)


# speedup vs baseline: 1.0030x; 1.0030x over previous
import jax
import jax.numpy as jnp
from jax import lax
from jax.experimental import pallas as pl
from jax.experimental.pallas import tpu as pltpu
from jax.experimental.pallas import tpu_sc as plsc

N = 10000
D = 128
G = 64
E = 320000

NP = 10112
TILES = 16
K = 120
T = 168
EP = TILES * T * K
NROW = 2 * EP // K
TROW = T
NPT = NP // TILES
BLK = 632
NBLK = (2 * NP) // BLK

ROWB = K * D * 4
IDXB = K * 4


def _memset_rows(ref, rows, cols, val):
    v = jnp.full((16,), val, jnp.float32)

    def body(i, _):
        r = i // (cols // 16)
        c = i % (cols // 16)
        ref[r, pl.ds(c * 16, 16)] = v
        return 0

    lax.fori_loop(0, rows * (cols // 16), body, 0)


def _sc_mesh():
    return plsc.VectorSubcoreMesh(core_axis_name="c", subcore_axis_name="s")


def _deg_body(dst_hbm, deg_hbm, shared_deg, didx, ones_v, zero_v,
              is0, is1, is2, is3, is4, is5):
    c = lax.axis_index("c")
    s = lax.axis_index("s")
    isem = [is0, is1, is2, is3, is4, is5]

    _memset_rows(ones_v, K, 16, 1.0)
    _memset_rows(zero_v, NPT, 16, 0.0)
    pltpu.sync_copy(zero_v, shared_deg.at[pl.ds(s * NPT, NPT)])
    plsc.subcore_barrier()

    base = c * EP + s * TROW * K

    def load_idx(j, u):
        pltpu.async_copy(dst_hbm.at[pl.ds(base + j * K, K)], didx.at[u],
                         isem[u])

    def wait_idx(u):
        pltpu.make_async_copy(dst_hbm.at[pl.ds(0, K)], didx.at[u],
                              isem[u]).wait()

    for j in range(4):
        load_idx(j, j % 6)

    def body(it, _):
        j0 = it * 6
        for u in range(6):
            j = j0 + u
            wait_idx(u)
            pltpu.sync_copy(ones_v, shared_deg.at[didx.at[u]], add=True)

            @pl.when(j + 4 < T)
            def _():
                load_idx(j + 4, (u + 4) % 6)
        return 0

    lax.fori_loop(0, T // 6, body, 0)
    plsc.subcore_barrier()

    pltpu.sync_copy(shared_deg.at[pl.ds(s * NPT, NPT)],
                    deg_hbm.at[pl.ds(c * NP + s * NPT, NPT)])


_deg_call = pl.kernel(
    _deg_body,
    out_type=jax.ShapeDtypeStruct((2 * NP, 16), jnp.float32),
    mesh=_sc_mesh(),
    scratch_types=[
        pltpu.VMEM_SHARED((NP, 16), jnp.float32),
        pltpu.VMEM((6, K), jnp.int32),
        pltpu.VMEM((K, 16), jnp.float32),
        pltpu.VMEM((NPT, 16), jnp.float32),
    ] + [pltpu.SemaphoreType.DMA] * 6,
)


def _agg_body(g_hbm, src_hbm, dst_hbm, agg_hbm,
              shared_agg, rows0, rows1, rows2, sidx, didx,
              gs0, gs1, gs2, ss0, ss1, ss2,
              is0, is1, is2, is3, is4, is5,
              js0, js1, js2, js3, js4, js5):
    c = lax.axis_index("c")
    s = lax.axis_index("s")
    rows = [rows0, rows1, rows2]
    gsem = [gs0, gs1, gs2]
    ssem = [ss0, ss1, ss2]
    isem = [is0, is1, is2, is3, is4, is5]
    jsem = [js0, js1, js2, js3, js4, js5]

    _memset_rows(rows0, K, D, 0.0)
    for t in range(5):
        pltpu.sync_copy(rows0, shared_agg.at[pl.ds(s * NPT + t * K, K)])
    pltpu.sync_copy(rows0, shared_agg.at[pl.ds(s * NPT + NPT - K, K)])
    plsc.subcore_barrier()

    base = c * EP + s * TROW * K

    def load_idx(j, u):
        pltpu.async_copy(src_hbm.at[pl.ds(base + j * K, K)], sidx.at[u],
                         isem[u])
        pltpu.async_copy(dst_hbm.at[pl.ds(base + j * K, K)], didx.at[u],
                         jsem[u])

    def wait_idx(u):
        pltpu.make_async_copy(src_hbm.at[pl.ds(0, K)], sidx.at[u],
                              isem[u]).wait()
        pltpu.make_async_copy(src_hbm.at[pl.ds(0, K)], didx.at[u],
                              jsem[u]).wait()

    def start_gather(u, u3):
        pltpu.async_copy(g_hbm.at[sidx.at[u]], rows[u3], gsem[u3])

    def wait_gather(u3):
        pltpu.make_async_copy(g_hbm.at[pl.ds(0, K)], rows[u3],
                              gsem[u3]).wait()

    def start_scatter(u, u3):
        pltpu.async_copy(rows[u3], shared_agg.at[didx.at[u]],
                         ssem[u3], add=True)

    def wait_scatter(u3):
        pltpu.make_async_copy(g_hbm.at[pl.ds(0, K)], rows[u3],
                              ssem[u3]).wait()

    for j in range(4):
        load_idx(j, j)
    wait_idx(0)
    start_gather(0, 0)
    wait_idx(1)
    start_gather(1, 1)

    def body(it, _):
        j0 = it * 6
        for u in range(6):
            j = j0 + u

            @pl.when(j >= 1)
            def _():
                wait_scatter((u + 2) % 3)

            @pl.when(j + 2 < T)
            def _():
                wait_idx((u + 2) % 6)
                start_gather((u + 2) % 6, (u + 2) % 3)

            @pl.when(j + 4 < T)
            def _():
                load_idx(j + 4, (u + 4) % 6)

            wait_gather(u % 3)
            start_scatter(u, u % 3)
        return 0

    lax.fori_loop(0, T // 6, body, 0)
    wait_scatter((T - 1) % 3)
    plsc.subcore_barrier()

    pltpu.sync_copy(shared_agg.at[pl.ds(s * NPT, NPT)],
                    agg_hbm.at[pl.ds(c * NP + s * NPT, NPT)])


_agg_call = pl.kernel(
    _agg_body,
    out_type=jax.ShapeDtypeStruct((2 * NP, D), jnp.float32),
    mesh=_sc_mesh(),
    scratch_types=[
        pltpu.VMEM_SHARED((NP, D), jnp.float32),
        pltpu.VMEM((K, D), jnp.float32),
        pltpu.VMEM((K, D), jnp.float32),
        pltpu.VMEM((K, D), jnp.float32),
        pltpu.VMEM((6, K), jnp.int32),
        pltpu.VMEM((6, K), jnp.int32),
    ] + [pltpu.SemaphoreType.DMA] * 18,
)


def _dis_block(deg_ref):
    d0 = deg_ref[...][:, 0:1] + 1.0
    return lax.rsqrt(d0)


def _prep_body(x_ref, deg_ref, w_ref, g_ref):
    dis = _dis_block(deg_ref)
    g_ref[...] = jnp.dot(x_ref[...], w_ref[...],
                         preferred_element_type=jnp.float32) * dis


_prep_call = pl.pallas_call(
    _prep_body,
    grid=(NBLK,),
    in_specs=[
        pl.BlockSpec((BLK, D), lambda i: (i, 0)),
        pl.BlockSpec((BLK, 16), lambda i: (i, 0)),
        pl.BlockSpec((D, D), lambda i: (0, 0)),
    ],
    out_specs=pl.BlockSpec((BLK, D), lambda i: (i, 0)),
    out_shape=jax.ShapeDtypeStruct((2 * NP, D), jnp.float32),
)


def _layer_body(agg_ref, g_ref, deg_ref, w_ref, b_ref, out_ref):
    dis = _dis_block(deg_ref)
    h = (agg_ref[...] + g_ref[...]) * dis + b_ref[...]
    h = jnp.maximum(h, 0.0)
    out_ref[...] = jnp.dot(h, w_ref[...],
                           preferred_element_type=jnp.float32) * dis


_layer_call = pl.pallas_call(
    _layer_body,
    grid=(NBLK,),
    in_specs=[
        pl.BlockSpec((BLK, D), lambda i: (i, 0)),
        pl.BlockSpec((BLK, D), lambda i: (i, 0)),
        pl.BlockSpec((BLK, 16), lambda i: (i, 0)),
        pl.BlockSpec((D, D), lambda i: (0, 0)),
        pl.BlockSpec((1, D), lambda i: (0, 0)),
    ],
    out_specs=pl.BlockSpec((BLK, D), lambda i: (i, 0)),
    out_shape=jax.ShapeDtypeStruct((2 * NP, D), jnp.float32),
)


def _final_body(agg_ref, g_ref, deg_ref, b_ref, batch_ref, wl_ref, bl_ref,
                out_ref, acc, cnt):
    i = pl.program_id(0)

    @pl.when(i == 0)
    def _():
        acc[...] = jnp.zeros_like(acc)
        cnt[...] = jnp.zeros_like(cnt)

    dis = _dis_block(deg_ref)
    h = (agg_ref[...] + g_ref[...]) * dis + b_ref[...]

    b2 = batch_ref[...].reshape(1, BLK)
    onehot = (lax.broadcasted_iota(jnp.int32, (G, BLK), 0) == b2)
    onehot = onehot.astype(jnp.float32)

    part = jnp.dot(onehot, h, preferred_element_type=jnp.float32)
    c = i // (NBLK // 2)
    acc[:, pl.ds(c * D, D)] += part
    cnt[:, pl.ds(c * D, D)] += jnp.sum(onehot, axis=1, keepdims=True)

    @pl.when(i == NBLK - 1)
    def _():
        pooled = acc[...] / jnp.maximum(cnt[...], 1.0)
        out_ref[...] = jnp.dot(pooled, wl_ref[...],
                               preferred_element_type=jnp.float32) + bl_ref[...]


_final_call = pl.pallas_call(
    _final_body,
    grid=(NBLK,),
    in_specs=[
        pl.BlockSpec((BLK, D), lambda i: (i, 0)),
        pl.BlockSpec((BLK, D), lambda i: (i, 0)),
        pl.BlockSpec((BLK, 16), lambda i: (i, 0)),
        pl.BlockSpec((1, D), lambda i: (0, 0)),
        pl.BlockSpec((1, 1, BLK), lambda i: (i, 0, 0)),
        pl.BlockSpec((2 * D, 2), lambda i: (0, 0)),
        pl.BlockSpec((1, 2), lambda i: (0, 0)),
    ],
    out_specs=pl.BlockSpec((G, 2), lambda i: (0, 0)),
    out_shape=jax.ShapeDtypeStruct((G, 2), jnp.float32),
    scratch_shapes=[
        pltpu.VMEM((G, 2 * D), jnp.float32),
        pltpu.VMEM((G, 2 * D), jnp.float32),
    ],
)


def kernel(x0, edge_attr0, edge_index0, x1, edge_attr1, edge_index1,
           batch0, batch1, W1, b1, W2, b2, W3, b3, WL, bL):
    npad = EP - E
    pad_rows = N + (jnp.arange(npad, dtype=jnp.int32) % (NP - N))

    src0 = jnp.concatenate([edge_index0[0], pad_rows])
    dst0 = jnp.concatenate([edge_index0[1], pad_rows])
    src1 = jnp.concatenate([edge_index1[0], pad_rows]) + NP
    dst1 = jnp.concatenate([edge_index1[1], pad_rows])
    src_all = jnp.concatenate([src0, src1])
    dst_all = jnp.concatenate([dst0, dst1])

    zrows = jnp.zeros((NP - N, D), jnp.float32)
    x_pad = jnp.concatenate([x0, zrows, x1, zrows])

    bpad = jnp.full((NP - N,), G, jnp.int32)
    batch_all = jnp.concatenate([batch0, bpad, batch1, bpad])
    batch3d = batch_all.reshape(NBLK, 1, BLK)

    b1r = b1.reshape(1, D)
    b2r = b2.reshape(1, D)
    b3r = b3.reshape(1, D)
    bLr = bL.reshape(1, 2)

    deg16 = _deg_call(dst_all)

    g1 = _prep_call(x_pad, deg16, W1)
    agg1 = _agg_call(g1, src_all, dst_all)
    g2 = _layer_call(agg1, g1, deg16, W2, b1r)
    agg2 = _agg_call(g2, src_all, dst_all)
    g3 = _layer_call(agg2, g2, deg16, W3, b2r)
    agg3 = _agg_call(g3, src_all, dst_all)
    return _final_call(agg3, g3, deg16, b3r, batch3d, WL, bLr)
